# Initial kernel scaffold; baseline (speedup 1.0000x reference)
import functools

import jax
import jax.numpy as jnp
from jax import lax
from jax.experimental import pallas as pl
from jax.experimental.pallas import tpu as pltpu
from jax.experimental.pallas import tpu_sc as plsc

_BN_EPS = 1e-5
_NSC = 2
_NTILES = 16
_NW = _NSC * _NTILES
_ROWS = 128


@functools.lru_cache(maxsize=None)
def _make_segsum(n, d, k, npad):
    rpt = npad // _NTILES
    n_zero = rpt // _ROWS

    def body(table_hbm, src_hbm, dst_hbm, out_hbm, src_v, dst_v, rows_v, acc,
             gsem):
        cid = lax.axis_index("c")
        sid = lax.axis_index("s")
        wid = sid * _NSC + cid

        zvec = jnp.zeros((16,), jnp.float32)

        def zrow(i, carry):
            for l in range(d // 16):
                rows_v[i, pl.ds(l * 16, 16)] = zvec
            return carry

        lax.fori_loop(0, _ROWS, zrow, 0)
        for z in range(n_zero):
            pltpu.sync_copy(rows_v, acc.at[pl.ds(sid * rpt + z * _ROWS, _ROWS)])
        plsc.subcore_barrier()

        pltpu.sync_copy(src_hbm.at[wid], src_v)
        pltpu.sync_copy(dst_hbm.at[wid], dst_v)

        def chunk(j, carry):
            pltpu.async_copy(table_hbm.at[src_v.at[j]], rows_v, gsem).wait()
            pltpu.sync_copy(rows_v, acc.at[dst_v.at[j]], add=True)
            return carry

        lax.fori_loop(0, k, chunk, 0)
        plsc.subcore_barrier()

        pltpu.sync_copy(acc.at[pl.ds(sid * rpt, rpt)],
                        out_hbm.at[cid, pl.ds(sid * rpt, rpt)])

    return pl.kernel(
        body,
        out_type=jax.ShapeDtypeStruct((_NSC, npad, d), jnp.float32),
        mesh=plsc.VectorSubcoreMesh(core_axis_name="c", subcore_axis_name="s"),
        scratch_types=[
            pltpu.VMEM((k, _ROWS), jnp.int32),
            pltpu.VMEM((k, _ROWS), jnp.int32),
            pltpu.VMEM((_ROWS, d), jnp.float32),
            pltpu.VMEM_SHARED((npad, d), jnp.float32),
            pltpu.SemaphoreType.DMA,
        ],
    )


def _segsum(table, src3, dst3, npad):
    n, d = table.shape
    k = src3.shape[1]
    return _make_segsum(n, d, k, npad)(table, src3, dst3)


def _mlp1_body(x_ref, parts_ref, w1_ref, b1_ref, g1_ref, bb1_ref, w2_ref,
               b2_ref, o_ref):
    h = x_ref[...] + parts_ref[0] + parts_ref[1]
    t = lax.dot_general(h, w1_ref[...], (((1,), (1,)), ((), ())),
                        preferred_element_type=jnp.float32)
    t = t + b1_ref[...]
    s = g1_ref[...] / jnp.sqrt(1.0 + _BN_EPS)
    t = jnp.maximum(t * s + bb1_ref[...], 0.0)
    t = lax.dot_general(t, w2_ref[...], (((1,), (1,)), ((), ())),
                        preferred_element_type=jnp.float32)
    o_ref[...] = jnp.maximum(t + b2_ref[...], 0.0)


def _mlp2_body(nb, h_ref, parts_ref, w_ref, b_ref, g_ref, bb_ref, batch_ref,
               o_ref):
    i = pl.program_id(0)
    h = h_ref[...] + parts_ref[0] + parts_ref[1]
    t = lax.dot_general(h, w_ref[...], (((1,), (1,)), ((), ())),
                        preferred_element_type=jnp.float32)
    s = g_ref[...] / jnp.sqrt(1.0 + _BN_EPS)
    hh = jnp.maximum((t + b_ref[...]) * s + bb_ref[...], 0.0)
    bi = batch_ref[0, 0, :]
    onehot = (lax.broadcasted_iota(jnp.int32, (nb, bi.shape[0]), 0)
              == bi[None, :]).astype(jnp.float32)
    acc = lax.dot_general(onehot, hh, (((1,), (0,)), ((), ())),
                          preferred_element_type=jnp.float32)

    @pl.when(i == 0)
    def _():
        o_ref[...] = jnp.zeros_like(o_ref)

    o_ref[...] += acc


def _sigma_max(w, n_square):
    a = lax.dot_general(w, w, (((1,), (1,)), ((), ())),
                        preferred_element_type=jnp.float32)

    def sq(_, b):
        b = b * lax.rsqrt(jnp.sum(b * b))
        return lax.dot_general(b, b, (((1,), (0,)), ((), ())),
                               preferred_element_type=jnp.float32)

    b = lax.fori_loop(0, n_square, sq, a * lax.rsqrt(jnp.sum(a * a)))
    ba = lax.dot_general(b, a, (((1,), (0,)), ((), ())),
                         preferred_element_type=jnp.float32)
    lam = jnp.sum(ba * b) / jnp.sum(b * b)
    return jnp.sqrt(lam)


def _head_body(nb, nf, p, g_struct_ref, dgx_ref, dgy_ref, msk_ref, w1x_ref,
               w1y_ref, pb1_ref, w2_ref, pb2_ref, rw1_ref, rb1_ref, rw2_ref,
               rb2_ref, f1w_ref, f1b_ref, f2w_ref, f2b_ref, o_ref):
    feats = []
    for f in range(nf):
        x1 = (dgx_ref[f][:, None] * w1x_ref[f][None, :]
              + dgy_ref[f][:, None] * w1y_ref[f][None, :]
              + pb1_ref[f][None, :])
        x1 = jnp.maximum(x1, 0.0)
        x2 = lax.dot_general(x1, w2_ref[f], (((1,), (1,)), ((), ())),
                             preferred_element_type=jnp.float32)
        x2 = x2 + pb2_ref[f][None, :]
        x2 = jnp.where(msk_ref[f][:, None] > 0.5, x2, -jnp.inf)
        m = jnp.max(x2.reshape(nb, p, x2.shape[1]), axis=1)
        feats.append(jnp.where(m == -jnp.inf, 0.0, m))
    feat = jnp.concatenate(feats, axis=1)
    r = jnp.maximum(
        lax.dot_general(feat, rw1_ref[...], (((1,), (1,)), ((), ())),
                        preferred_element_type=jnp.float32) + rb1_ref[...], 0.0)
    gp = lax.dot_general(r, rw2_ref[...], (((1,), (1,)), ((), ())),
                         preferred_element_type=jnp.float32) + rb2_ref[...]

    sig1 = _sigma_max(f1w_ref[...], 12)
    sig2 = _sigma_max(f2w_ref[...], 12)

    g = jnp.concatenate([g_struct_ref[...], gp], axis=1)
    z = lax.dot_general(g, f1w_ref[...], (((1,), (1,)), ((), ())),
                        preferred_element_type=jnp.float32) / sig1 + f1b_ref[...]
    e = jnp.where(z > 0.0, z, jnp.exp(z) - 1.0)
    out = lax.dot_general(e, f2w_ref[...], (((1,), (1,)), ((), ())),
                          preferred_element_type=jnp.float32) / sig2 + f2b_ref[...]
    o_ref[...] = out


def kernel(x, edge_index, batch, diagrams_batch, masks_batch, params):
    n, d = x.shape
    e = edge_index.shape[1]
    nb, nf, p, _ = diagrams_batch.shape
    h_dim = params["c1_W1"].shape[0]

    npad = 2048 * ((n + _ROWS + 2047) // 2048)
    k = -(-e // (_NW * _ROWS))
    k += k % 2
    ep = _NW * _ROWS * k
    pad = ep - e
    idx_pad = jnp.arange(pad, dtype=jnp.int32)
    src_p = jnp.concatenate([edge_index[0], idx_pad % n])
    dst_p = jnp.concatenate([edge_index[1], n + idx_pad % (npad - n)])
    src3 = src_p.reshape(_NW, k, _ROWS)
    dst3 = dst_p.reshape(_NW, k, _ROWS)

    parts1 = _segsum(x, src3, dst3, npad)

    r = 1000
    grid = (n // r,)
    full = lambda i: (0, 0)
    row_spec = pl.BlockSpec((r, d), lambda i: (i, 0))
    parts_spec = pl.BlockSpec((_NSC, r, d), lambda i: (0, i, 0))
    vec = lambda: pl.BlockSpec((1, h_dim), full)
    h1 = pl.pallas_call(
        _mlp1_body,
        grid=grid,
        in_specs=[row_spec, parts_spec,
                  pl.BlockSpec((h_dim, d), full), vec(), vec(), vec(),
                  pl.BlockSpec((h_dim, h_dim), full), vec()],
        out_specs=pl.BlockSpec((r, h_dim), lambda i: (i, 0)),
        out_shape=jax.ShapeDtypeStruct((n, h_dim), jnp.float32),
    )(x, parts1, params["c1_W1"], params["c1_b1"].reshape(1, -1),
      params["c1_bn_g"].reshape(1, -1), params["c1_bn_b"].reshape(1, -1),
      params["c1_W2"], params["c1_b2"].reshape(1, -1))

    parts2 = _segsum(h1, src3, dst3, npad)

    batch3 = batch.reshape(n // r, 1, r)
    g_struct = pl.pallas_call(
        functools.partial(_mlp2_body, nb),
        grid=grid,
        in_specs=[row_spec, parts_spec,
                  pl.BlockSpec((h_dim, h_dim), full), vec(), vec(), vec(),
                  pl.BlockSpec((1, 1, r), lambda i: (i, 0, 0))],
        out_specs=pl.BlockSpec((nb, h_dim), full),
        out_shape=jax.ShapeDtypeStruct((nb, h_dim), jnp.float32),
    )(h1, parts2, params["c2_W"], params["c2_b"].reshape(1, -1),
      params["c2_bn_g"].reshape(1, -1), params["c2_bn_b"].reshape(1, -1),
      batch3)

    diag = diagrams_batch.transpose(1, 0, 2, 3).reshape(nf, nb * p, 2)
    dgx = diag[:, :, 0]
    dgy = diag[:, :, 1]
    msk = masks_batch.transpose(1, 0, 2).reshape(nf, nb * p).astype(jnp.float32)
    nk = params["pl_W1"].shape[1]
    nc = params["fc2_W"].shape[0]
    w1x = params["pl_W1"][:, :, 0]
    w1y = params["pl_W1"][:, :, 1]

    bp_spec = pl.BlockSpec((nf, nb * p), full)
    k_spec = pl.BlockSpec((nf, nk), full)
    out = pl.pallas_call(
        functools.partial(_head_body, nb, nf, p),
        in_specs=[pl.BlockSpec((nb, h_dim), full),
                  bp_spec, bp_spec, bp_spec,
                  k_spec, k_spec, k_spec,
                  pl.BlockSpec((nf, nk, nk), lambda i=None: (0, 0, 0)),
                  k_spec,
                  pl.BlockSpec((h_dim, nf * nk), full), vec(),
                  pl.BlockSpec((h_dim, h_dim), full), vec(),
                  pl.BlockSpec((h_dim, 2 * h_dim), full), vec(),
                  pl.BlockSpec((nc, h_dim), full),
                  pl.BlockSpec((1, nc), full)],
        out_specs=pl.BlockSpec((nb, nc), full),
        out_shape=jax.ShapeDtypeStruct((nb, nc), jnp.float32),
    )(g_struct, dgx, dgy, msk, w1x, w1y, params["pl_b1"], params["pl_W2"],
      params["pl_b2"], params["rho_W1"], params["rho_b1"].reshape(1, -1),
      params["rho_W2"], params["rho_b2"].reshape(1, -1), params["fc1_W"],
      params["fc1_b"].reshape(1, -1), params["fc2_W"],
      params["fc2_b"].reshape(1, -1))
    return out


# speedup vs baseline: 8.3097x; 8.3097x over previous
import functools

import jax
import jax.numpy as jnp
from jax import lax
from jax.experimental import pallas as pl
from jax.experimental.pallas import tpu as pltpu
from jax.experimental.pallas import tpu_sc as plsc

_BN_EPS = 1e-5
_NSC = 2
_NTILES = 16
_NW = _NSC * _NTILES
_ROWS = 128


@functools.lru_cache(maxsize=None)
def _make_segsum(n, d, k, npad):
    rpt = npad // _NTILES
    n_zero = rpt // _ROWS

    def body(table_hbm, src_hbm, dst_hbm, out_hbm, src_v, dst_v, rows_v, acc,
             gsem):
        cid = lax.axis_index("c")
        sid = lax.axis_index("s")
        wid = sid * _NSC + cid

        zvec = jnp.zeros((16,), jnp.float32)

        def zrow(i, carry):
            for l in range(d // 16):
                rows_v[i, pl.ds(l * 16, 16)] = zvec
            return carry

        lax.fori_loop(0, _ROWS, zrow, 0)
        for z in range(n_zero):
            pltpu.sync_copy(rows_v, acc.at[pl.ds(sid * rpt + z * _ROWS, _ROWS)])
        plsc.subcore_barrier()

        pltpu.sync_copy(src_hbm.at[wid], src_v)
        pltpu.sync_copy(dst_hbm.at[wid], dst_v)

        def chunk(j, carry):
            pltpu.async_copy(table_hbm.at[src_v.at[j]], rows_v, gsem).wait()
            pltpu.sync_copy(rows_v, acc.at[dst_v.at[j]], add=True)
            return carry

        lax.fori_loop(0, k, chunk, 0)
        plsc.subcore_barrier()

        pltpu.sync_copy(acc.at[pl.ds(sid * rpt, rpt)],
                        out_hbm.at[cid, pl.ds(sid * rpt, rpt)])

    return pl.kernel(
        body,
        out_type=jax.ShapeDtypeStruct((_NSC, npad, d), jnp.float32),
        mesh=plsc.VectorSubcoreMesh(core_axis_name="c", subcore_axis_name="s"),
        scratch_types=[
            pltpu.VMEM((k, _ROWS), jnp.int32),
            pltpu.VMEM((k, _ROWS), jnp.int32),
            pltpu.VMEM((_ROWS, d), jnp.float32),
            pltpu.VMEM_SHARED((npad, d), jnp.float32),
            pltpu.SemaphoreType.DMA,
        ],
    )


def _segsum(table, src3, dst3, npad):
    n, d = table.shape
    k = src3.shape[1]
    return _make_segsum(n, d, k, npad)(table, src3, dst3)


def _mlp1_body(x_ref, parts_ref, w1_ref, b1_ref, g1_ref, bb1_ref, w2_ref,
               b2_ref, o_ref):
    h = x_ref[...] + parts_ref[0] + parts_ref[1]
    t = lax.dot_general(h, w1_ref[...], (((1,), (1,)), ((), ())),
                        preferred_element_type=jnp.float32)
    t = t + b1_ref[...]
    s = g1_ref[...] / jnp.sqrt(1.0 + _BN_EPS)
    t = jnp.maximum(t * s + bb1_ref[...], 0.0)
    t = lax.dot_general(t, w2_ref[...], (((1,), (1,)), ((), ())),
                        preferred_element_type=jnp.float32)
    o_ref[...] = jnp.maximum(t + b2_ref[...], 0.0)


def _mlp2_body(nb, h_ref, parts_ref, w_ref, b_ref, g_ref, bb_ref, batch_ref,
               o_ref):
    i = pl.program_id(0)
    h = h_ref[...] + parts_ref[0] + parts_ref[1]
    t = lax.dot_general(h, w_ref[...], (((1,), (1,)), ((), ())),
                        preferred_element_type=jnp.float32)
    s = g_ref[...] / jnp.sqrt(1.0 + _BN_EPS)
    hh = jnp.maximum((t + b_ref[...]) * s + bb_ref[...], 0.0)
    bi = batch_ref[0, 0, :]
    onehot = (lax.broadcasted_iota(jnp.int32, (nb, bi.shape[0]), 0)
              == bi[None, :]).astype(jnp.float32)
    acc = lax.dot_general(onehot, hh, (((1,), (0,)), ((), ())),
                          preferred_element_type=jnp.float32,
                          precision=lax.Precision.HIGHEST)

    @pl.when(i == 0)
    def _():
        o_ref[...] = jnp.zeros_like(o_ref)

    o_ref[...] += acc


def _sigma_max(w, n_square):
    a = lax.dot_general(w, w, (((1,), (1,)), ((), ())),
                        preferred_element_type=jnp.float32,
                        precision=lax.Precision.HIGHEST)

    b = a * lax.rsqrt(jnp.sum(a * a))
    for _ in range(n_square):
        b = lax.dot_general(b, b, (((1,), (0,)), ((), ())),
                            preferred_element_type=jnp.float32,
                            precision=lax.Precision.HIGHEST)
        b = b * lax.rsqrt(jnp.sum(b * b))
    ba = lax.dot_general(b, a, (((1,), (0,)), ((), ())),
                         preferred_element_type=jnp.float32,
                         precision=lax.Precision.HIGHEST)
    lam = jnp.sum(ba * b) / jnp.sum(b * b)
    return jnp.sqrt(lam)


def _head_body(nb, nf, p, g_struct_ref, dgx_ref, dgy_ref, msk_ref, w1x_ref,
               w1y_ref, pb1_ref, w2_ref, pb2_ref, rw1_ref, rb1_ref, rw2_ref,
               rb2_ref, f1w_ref, f1b_ref, f2w_ref, f2b_ref, o_ref):
    feats = []
    for f in range(nf):
        x1 = (dgx_ref[f][:, None] * w1x_ref[f][None, :]
              + dgy_ref[f][:, None] * w1y_ref[f][None, :]
              + pb1_ref[f][None, :])
        x1 = jnp.maximum(x1, 0.0)
        x2 = lax.dot_general(x1, w2_ref[f], (((1,), (1,)), ((), ())),
                             preferred_element_type=jnp.float32)
        x2 = x2 + pb2_ref[f][None, :]
        x2 = jnp.where(msk_ref[f][:, None] > 0.5, x2, -jnp.inf)
        m = jnp.max(x2.reshape(nb, p, x2.shape[1]), axis=1)
        feats.append(jnp.where(m == -jnp.inf, 0.0, m))
    feat = jnp.concatenate(feats, axis=1)
    r = jnp.maximum(
        lax.dot_general(feat, rw1_ref[...], (((1,), (1,)), ((), ())),
                        preferred_element_type=jnp.float32) + rb1_ref[...], 0.0)
    gp = lax.dot_general(r, rw2_ref[...], (((1,), (1,)), ((), ())),
                         preferred_element_type=jnp.float32) + rb2_ref[...]

    sig1 = _sigma_max(f1w_ref[...], 12)
    w0 = f2w_ref[0, :]
    w1 = f2w_ref[1, :]
    ga = jnp.sum(w0 * w0)
    gb = jnp.sum(w0 * w1)
    gc = jnp.sum(w1 * w1)
    lam2 = 0.5 * (ga + gc) + jnp.sqrt(0.25 * (ga - gc) ** 2 + gb * gb)
    sig2 = jnp.sqrt(lam2)

    g = jnp.concatenate([g_struct_ref[...], gp], axis=1)
    z = lax.dot_general(g, f1w_ref[...] / sig1, (((1,), (1,)), ((), ())),
                        preferred_element_type=jnp.float32) + f1b_ref[...]
    e = jnp.where(z > 0.0, z, jnp.exp(z) - 1.0)
    out = lax.dot_general(e, f2w_ref[...] / sig2, (((1,), (1,)), ((), ())),
                          preferred_element_type=jnp.float32) + f2b_ref[...]
    o_ref[...] = out


def kernel(x, edge_index, batch, diagrams_batch, masks_batch, params):
    n, d = x.shape
    e = edge_index.shape[1]
    nb, nf, p, _ = diagrams_batch.shape
    h_dim = params["c1_W1"].shape[0]

    npad = 2048 * ((n + _ROWS + 2047) // 2048)
    k = -(-e // (_NW * _ROWS))
    k += k % 2
    ep = _NW * _ROWS * k
    pad = ep - e
    idx_pad = jnp.arange(pad, dtype=jnp.int32)
    src_p = jnp.concatenate([edge_index[0], idx_pad % n])
    dst_p = jnp.concatenate([edge_index[1], n + idx_pad % (npad - n)])
    src3 = src_p.reshape(_NW, k, _ROWS)
    dst3 = dst_p.reshape(_NW, k, _ROWS)

    parts1 = _segsum(x, src3, dst3, npad)

    r = 1000
    grid = (n // r,)
    full = lambda i: (0, 0)
    row_spec = pl.BlockSpec((r, d), lambda i: (i, 0))
    parts_spec = pl.BlockSpec((_NSC, r, d), lambda i: (0, i, 0))
    vec = lambda: pl.BlockSpec((1, h_dim), full)
    h1 = pl.pallas_call(
        _mlp1_body,
        grid=grid,
        in_specs=[row_spec, parts_spec,
                  pl.BlockSpec((h_dim, d), full), vec(), vec(), vec(),
                  pl.BlockSpec((h_dim, h_dim), full), vec()],
        out_specs=pl.BlockSpec((r, h_dim), lambda i: (i, 0)),
        out_shape=jax.ShapeDtypeStruct((n, h_dim), jnp.float32),
    )(x, parts1, params["c1_W1"], params["c1_b1"].reshape(1, -1),
      params["c1_bn_g"].reshape(1, -1), params["c1_bn_b"].reshape(1, -1),
      params["c1_W2"], params["c1_b2"].reshape(1, -1))

    parts2 = _segsum(h1, src3, dst3, npad)

    batch3 = batch.reshape(n // r, 1, r)
    g_struct = pl.pallas_call(
        functools.partial(_mlp2_body, nb),
        grid=grid,
        in_specs=[row_spec, parts_spec,
                  pl.BlockSpec((h_dim, h_dim), full), vec(), vec(), vec(),
                  pl.BlockSpec((1, 1, r), lambda i: (i, 0, 0))],
        out_specs=pl.BlockSpec((nb, h_dim), full),
        out_shape=jax.ShapeDtypeStruct((nb, h_dim), jnp.float32),
    )(h1, parts2, params["c2_W"], params["c2_b"].reshape(1, -1),
      params["c2_bn_g"].reshape(1, -1), params["c2_bn_b"].reshape(1, -1),
      batch3)

    diag = diagrams_batch.transpose(1, 0, 2, 3).reshape(nf, nb * p, 2)
    dgx = diag[:, :, 0]
    dgy = diag[:, :, 1]
    msk = masks_batch.transpose(1, 0, 2).reshape(nf, nb * p).astype(jnp.float32)
    nk = params["pl_W1"].shape[1]
    nc = params["fc2_W"].shape[0]
    w1x = params["pl_W1"][:, :, 0]
    w1y = params["pl_W1"][:, :, 1]

    out = pl.pallas_call(
        functools.partial(_head_body, nb, nf, p),
        out_shape=jax.ShapeDtypeStruct((nb, nc), jnp.float32),
    )(g_struct, dgx, dgy, msk, w1x, w1y, params["pl_b1"], params["pl_W2"],
      params["pl_b2"], params["rho_W1"], params["rho_b1"].reshape(1, -1),
      params["rho_W2"], params["rho_b2"].reshape(1, -1), params["fc1_W"],
      params["fc1_b"].reshape(1, -1), params["fc2_W"],
      params["fc2_b"].reshape(1, -1))
    return out


# speedup vs baseline: 10.5376x; 1.2681x over previous
import functools

import jax
import jax.numpy as jnp
from jax import lax
from jax.experimental import pallas as pl
from jax.experimental.pallas import tpu as pltpu
from jax.experimental.pallas import tpu_sc as plsc

_BN_EPS = 1e-5
_NSC = 2
_NTILES = 16
_NW = _NSC * _NTILES
_ROWS = 128


@functools.lru_cache(maxsize=None)
def _make_segsum(n, d, k, npad):
    rpt = npad // _NTILES
    n_zero = rpt // _ROWS

    def body(table_hbm, src_hbm, dst_hbm, out_hbm, src_v, dst_v, rows_a,
             rows_b, acc, sem_a, sem_b):
        cid = lax.axis_index("c")
        sid = lax.axis_index("s")
        wid = sid * _NSC + cid

        zvec = jnp.zeros((16,), jnp.float32)

        def zrow(i, carry):
            for l in range(d // 16):
                rows_a[i, pl.ds(l * 16, 16)] = zvec
            return carry

        lax.fori_loop(0, _ROWS, zrow, 0)
        for z in range(n_zero):
            pltpu.sync_copy(rows_a, acc.at[pl.ds(sid * rpt + z * _ROWS, _ROWS)])
        plsc.subcore_barrier()

        w_rows = k // 2

        def wait_g(buf, sem):
            pltpu.make_async_copy(table_hbm.at[src_v.at[0]], buf, sem).wait()

        def window(w, carry):
            pltpu.sync_copy(src_hbm.at[wid, pl.ds(w * w_rows, w_rows)], src_v)
            pltpu.sync_copy(dst_hbm.at[wid, pl.ds(w * w_rows, w_rows)], dst_v)

            pltpu.async_copy(table_hbm.at[src_v.at[0]], rows_a, sem_a)

            def chunk2(jj, c2):
                j = 2 * jj
                wait_g(rows_a, sem_a)
                pltpu.async_copy(table_hbm.at[src_v.at[j + 1]], rows_b, sem_b)
                pltpu.sync_copy(rows_a, acc.at[dst_v.at[j]], add=True)
                wait_g(rows_b, sem_b)

                @pl.when(jj < w_rows // 2 - 1)
                def _():
                    pltpu.async_copy(table_hbm.at[src_v.at[j + 2]], rows_a,
                                     sem_a)

                pltpu.sync_copy(rows_b, acc.at[dst_v.at[j + 1]], add=True)
                return c2

            lax.fori_loop(0, w_rows // 2, chunk2, 0)
            return carry

        lax.fori_loop(0, 2, window, 0)
        plsc.subcore_barrier()

        pltpu.sync_copy(acc.at[pl.ds(sid * rpt, rpt)],
                        out_hbm.at[cid, pl.ds(sid * rpt, rpt)])

    return pl.kernel(
        body,
        out_type=jax.ShapeDtypeStruct((_NSC, npad, d), jnp.float32),
        mesh=plsc.VectorSubcoreMesh(core_axis_name="c", subcore_axis_name="s"),
        scratch_types=[
            pltpu.VMEM((k // 2, _ROWS), jnp.int32),
            pltpu.VMEM((k // 2, _ROWS), jnp.int32),
            pltpu.VMEM((_ROWS, d), jnp.float32),
            pltpu.VMEM((_ROWS, d), jnp.float32),
            pltpu.VMEM_SHARED((npad, d), jnp.float32),
            pltpu.SemaphoreType.DMA,
            pltpu.SemaphoreType.DMA,
        ],
    )


def _segsum(table, src3, dst3, npad):
    n, d = table.shape
    k = src3.shape[1]
    return _make_segsum(n, d, k, npad)(table, src3, dst3)


def _mlp1_body(x_ref, parts_ref, w1_ref, b1_ref, g1_ref, bb1_ref, w2_ref,
               b2_ref, o_ref):
    h = x_ref[...] + parts_ref[0] + parts_ref[1]
    t = lax.dot_general(h, w1_ref[...], (((1,), (1,)), ((), ())),
                        preferred_element_type=jnp.float32)
    t = t + b1_ref[...]
    s = g1_ref[...] / jnp.sqrt(1.0 + _BN_EPS)
    t = jnp.maximum(t * s + bb1_ref[...], 0.0)
    t = lax.dot_general(t, w2_ref[...], (((1,), (1,)), ((), ())),
                        preferred_element_type=jnp.float32)
    o_ref[...] = jnp.maximum(t + b2_ref[...], 0.0)


def _mlp2_body(nb, h_ref, parts_ref, w_ref, b_ref, g_ref, bb_ref, batch_ref,
               o_ref):
    i = pl.program_id(0)
    h = h_ref[...] + parts_ref[0] + parts_ref[1]
    t = lax.dot_general(h, w_ref[...], (((1,), (1,)), ((), ())),
                        preferred_element_type=jnp.float32)
    s = g_ref[...] / jnp.sqrt(1.0 + _BN_EPS)
    hh = jnp.maximum((t + b_ref[...]) * s + bb_ref[...], 0.0)
    bi = batch_ref[0, 0, :]
    onehot = (lax.broadcasted_iota(jnp.int32, (nb, bi.shape[0]), 0)
              == bi[None, :]).astype(jnp.float32)
    acc = lax.dot_general(onehot, hh, (((1,), (0,)), ((), ())),
                          preferred_element_type=jnp.float32,
                          precision=lax.Precision.HIGHEST)

    @pl.when(i == 0)
    def _():
        o_ref[...] = jnp.zeros_like(o_ref)

    o_ref[...] += acc


def _sigma_max(w, n_square):
    a = lax.dot_general(w, w, (((1,), (1,)), ((), ())),
                        preferred_element_type=jnp.float32,
                        precision=lax.Precision.HIGHEST)

    b = a * lax.rsqrt(jnp.sum(a * a))
    for _ in range(n_square):
        b = lax.dot_general(b, b, (((1,), (0,)), ((), ())),
                            preferred_element_type=jnp.float32,
                            precision=lax.Precision.HIGHEST)
        b = b * lax.rsqrt(jnp.sum(b * b))
    ba = lax.dot_general(b, a, (((1,), (0,)), ((), ())),
                         preferred_element_type=jnp.float32,
                         precision=lax.Precision.HIGHEST)
    lam = jnp.sum(ba * b) / jnp.sum(b * b)
    return jnp.sqrt(lam)


def _head_body(nb, nf, p, g_struct_ref, dgx_ref, dgy_ref, msk_ref, w1x_ref,
               w1y_ref, pb1_ref, w2_ref, pb2_ref, rw1_ref, rb1_ref, rw2_ref,
               rb2_ref, f1w_ref, f1b_ref, f2w_ref, f2b_ref, o_ref):
    feats = []
    for f in range(nf):
        x1 = (dgx_ref[f][:, None] * w1x_ref[f][None, :]
              + dgy_ref[f][:, None] * w1y_ref[f][None, :]
              + pb1_ref[f][None, :])
        x1 = jnp.maximum(x1, 0.0)
        x2 = lax.dot_general(x1, w2_ref[f], (((1,), (1,)), ((), ())),
                             preferred_element_type=jnp.float32)
        x2 = x2 + pb2_ref[f][None, :]
        x2 = jnp.where(msk_ref[f][:, None] > 0.5, x2, -jnp.inf)
        m = jnp.max(x2.reshape(nb, p, x2.shape[1]), axis=1)
        feats.append(jnp.where(m == -jnp.inf, 0.0, m))
    feat = jnp.concatenate(feats, axis=1)
    r = jnp.maximum(
        lax.dot_general(feat, rw1_ref[...], (((1,), (1,)), ((), ())),
                        preferred_element_type=jnp.float32) + rb1_ref[...], 0.0)
    gp = lax.dot_general(r, rw2_ref[...], (((1,), (1,)), ((), ())),
                         preferred_element_type=jnp.float32) + rb2_ref[...]

    sig1 = _sigma_max(f1w_ref[...], 12)
    w0 = f2w_ref[0, :]
    w1 = f2w_ref[1, :]
    ga = jnp.sum(w0 * w0)
    gb = jnp.sum(w0 * w1)
    gc = jnp.sum(w1 * w1)
    lam2 = 0.5 * (ga + gc) + jnp.sqrt(0.25 * (ga - gc) ** 2 + gb * gb)
    sig2 = jnp.sqrt(lam2)

    g = jnp.concatenate([g_struct_ref[...], gp], axis=1)
    z = lax.dot_general(g, f1w_ref[...] / sig1, (((1,), (1,)), ((), ())),
                        preferred_element_type=jnp.float32) + f1b_ref[...]
    e = jnp.where(z > 0.0, z, jnp.exp(z) - 1.0)
    out = lax.dot_general(e, f2w_ref[...] / sig2, (((1,), (1,)), ((), ())),
                          preferred_element_type=jnp.float32) + f2b_ref[...]
    o_ref[...] = out


def kernel(x, edge_index, batch, diagrams_batch, masks_batch, params):
    n, d = x.shape
    e = edge_index.shape[1]
    nb, nf, p, _ = diagrams_batch.shape
    h_dim = params["c1_W1"].shape[0]

    npad = 2048 * ((n + _ROWS + 2047) // 2048)
    k = -(-e // (_NW * _ROWS))
    k += (-k) % 4
    ep = _NW * _ROWS * k
    pad = ep - e
    idx_pad = jnp.arange(pad, dtype=jnp.int32)
    src_p = jnp.concatenate([edge_index[0], idx_pad % n])
    dst_p = jnp.concatenate([edge_index[1], n + idx_pad % (npad - n)])
    src3 = src_p.reshape(_NW, k, _ROWS)
    dst3 = dst_p.reshape(_NW, k, _ROWS)

    parts1 = _segsum(x, src3, dst3, npad)

    r = 1000
    grid = (n // r,)
    full = lambda i: (0, 0)
    row_spec = pl.BlockSpec((r, d), lambda i: (i, 0))
    parts_spec = pl.BlockSpec((_NSC, r, d), lambda i: (0, i, 0))
    vec = lambda: pl.BlockSpec((1, h_dim), full)
    h1 = pl.pallas_call(
        _mlp1_body,
        grid=grid,
        in_specs=[row_spec, parts_spec,
                  pl.BlockSpec((h_dim, d), full), vec(), vec(), vec(),
                  pl.BlockSpec((h_dim, h_dim), full), vec()],
        out_specs=pl.BlockSpec((r, h_dim), lambda i: (i, 0)),
        out_shape=jax.ShapeDtypeStruct((n, h_dim), jnp.float32),
    )(x, parts1, params["c1_W1"], params["c1_b1"].reshape(1, -1),
      params["c1_bn_g"].reshape(1, -1), params["c1_bn_b"].reshape(1, -1),
      params["c1_W2"], params["c1_b2"].reshape(1, -1))

    parts2 = _segsum(h1, src3, dst3, npad)

    batch3 = batch.reshape(n // r, 1, r)
    g_struct = pl.pallas_call(
        functools.partial(_mlp2_body, nb),
        grid=grid,
        in_specs=[row_spec, parts_spec,
                  pl.BlockSpec((h_dim, h_dim), full), vec(), vec(), vec(),
                  pl.BlockSpec((1, 1, r), lambda i: (i, 0, 0))],
        out_specs=pl.BlockSpec((nb, h_dim), full),
        out_shape=jax.ShapeDtypeStruct((nb, h_dim), jnp.float32),
    )(h1, parts2, params["c2_W"], params["c2_b"].reshape(1, -1),
      params["c2_bn_g"].reshape(1, -1), params["c2_bn_b"].reshape(1, -1),
      batch3)

    diag = diagrams_batch.transpose(1, 0, 2, 3).reshape(nf, nb * p, 2)
    dgx = diag[:, :, 0]
    dgy = diag[:, :, 1]
    msk = masks_batch.transpose(1, 0, 2).reshape(nf, nb * p).astype(jnp.float32)
    nk = params["pl_W1"].shape[1]
    nc = params["fc2_W"].shape[0]
    w1x = params["pl_W1"][:, :, 0]
    w1y = params["pl_W1"][:, :, 1]

    out = pl.pallas_call(
        functools.partial(_head_body, nb, nf, p),
        out_shape=jax.ShapeDtypeStruct((nb, nc), jnp.float32),
    )(g_struct, dgx, dgy, msk, w1x, w1y, params["pl_b1"], params["pl_W2"],
      params["pl_b2"], params["rho_W1"], params["rho_b1"].reshape(1, -1),
      params["rho_W2"], params["rho_b2"].reshape(1, -1), params["fc1_W"],
      params["fc1_b"].reshape(1, -1), params["fc2_W"],
      params["fc2_b"].reshape(1, -1))
    return out
